# baseline (speedup 1.0000x reference)
import functools

import jax
import jax.numpy as jnp
import numpy as np
from jax import lax
from jax.experimental import pallas as pl
from jax.experimental.pallas import tpu as pltpu
from jax.experimental.pallas import tpu_sc as plsc

B, N_NUM, N_CAT, D, V = 16384, 13, 26, 32, 1000000
NF = N_NUM + N_CAT
TOT = B * N_CAT
NW = 32
PER_W = TOT // NW
CH = 128
K = 8
GROUPS = PER_W // (CH * K)
NF_T = N_CAT * D // 8
BL = 512


def _sc_gather(table, idx_lin):
    mesh = plsc.VectorSubcoreMesh(core_axis_name="c", subcore_axis_name="s")

    @functools.partial(
        pl.kernel,
        mesh=mesh,
        out_type=jax.ShapeDtypeStruct((NF_T, 128, 8, 128), jnp.float32),
        scratch_types=[
            pltpu.VMEM((4, 4, 1024), jnp.int32),
            pltpu.VMEM((512, D), jnp.float32),
            pltpu.VMEM((4, 4, 8, 128), jnp.float32),
            pltpu.SemaphoreType.DMA,
        ],
        compiler_params=pltpu.CompilerParams(
            use_tc_tiling_on_sc=False, needs_layout_passes=False),
    )
    def gather_kernel(table_hbm, idx_hbm, out_hbm, idx_v, rows_v, tile_v, sem):
        wid = lax.axis_index("s") * 2 + lax.axis_index("c")
        for kb in range(4):
            for tc in range(4):
                pltpu.sync_copy(
                    idx_hbm.at[pl.ds((tc * 128 + wid * 4 + kb) * 8 * 128, 1024)],
                    idx_v.at[kb, tc],
                )
        iota = lax.iota(jnp.int32, 16)

        def per_feature(c, carry):
            tc = c // 8
            i = c % 8
            handles = []
            for kb in range(4):
                handles.append(
                    pltpu.async_copy(
                        table_hbm.at[idx_v.at[kb, tc, pl.ds(i * CH, CH)]],
                        rows_v.at[pl.ds(kb * CH, CH)],
                        sem,
                    )
                )
            for h in handles:
                h.wait()
            for kb in range(4):
                for d in range(D):
                    for k in range(8):
                        rows_idx = iota + (kb * CH + k * 16)
                        col_idx = jnp.full((16,), d, jnp.int32)
                        val = plsc.load_gather(rows_v, [rows_idx, col_idx])
                        tile_v[kb, d // 8, d % 8, pl.ds(k * 16, 16)] = val
            for kb in range(4):
                pltpu.sync_copy(
                    tile_v.at[kb],
                    out_hbm.at[pl.ds(4 * c, 4), wid * 4 + kb],
                )
            return carry

        lax.fori_loop(0, N_CAT, per_feature, 0)

    return gather_kernel(table, idx_lin)


def _tc_body(xT_r, idxT_r, metaT_r, embT_r, WnT_r, WmT_r, bn_r, bm_r,
             fid_r, g_r, bt_r, h0_o, raw_o, mask_o, um_o):
    f32 = jnp.float32
    gelu = lambda t: 0.5 * t * (1.0 + lax.erf(t * np.float32(0.7071067811865476)))

    x3 = xT_r[...].reshape(3, N_NUM, BL)
    m3 = metaT_r[...].reshape(N_CAT, 2, BL)
    e3 = embT_r[...].reshape(N_CAT, D, BL)
    idxv = idxT_r[...]
    fid = fid_r[...]
    gam = g_r[...][None]
    bet = bt_r[...][None]

    wcol = lambda ref, j: ref[:, j:j + 1][None]

    zn = (x3[0][:, None, :] * wcol(WnT_r, 0)
          + x3[1][:, None, :] * wcol(WnT_r, 1)
          + x3[2][:, None, :] * wcol(WnT_r, 2)
          + bn_r[...][None])
    zn = gelu(zn) + fid[:N_NUM]

    zc = (e3
          + m3[:, 0:1, :] * wcol(WmT_r, 0)
          + m3[:, 1:2, :] * wcol(WmT_r, 1)
          + bm_r[...][None])
    zc = gelu(zc) + fid[N_NUM:]

    def ln(z):
        mean = jnp.mean(z, axis=1, keepdims=True)
        var = jnp.mean(z * z, axis=1, keepdims=True) - mean * mean
        return (z - mean) * lax.rsqrt(var + np.float32(1e-5)) * gam + bet

    h0_o[:N_NUM * D, :] = ln(zn).reshape(N_NUM * D, BL)
    h0_o[N_NUM * D:, :] = ln(zc).reshape(N_CAT * D, BL)

    idx_f = idxv.astype(f32)
    raw_o[...] = jnp.concatenate([x3[0], idx_f], axis=0)

    mask_o[...] = jnp.where(
        lax.broadcasted_iota(jnp.int32, (NF, BL), 0) < N_NUM,
        np.float32(1.0), np.float32(0.0))

    um_num = jnp.concatenate(
        [x3[1][:, None, :], x3[2][:, None, :]], axis=1).reshape(2 * N_NUM, BL)
    sign = (idxv % 2 * 2 - 1).astype(f32)
    tf = 0.5 + sign * (0.5 - 0.5 * m3[:, 0, :])
    um_cat = jnp.concatenate(
        [tf[:, None, :], m3[:, 1, :][:, None, :]], axis=1).reshape(2 * N_CAT, BL)
    um_o[...] = jnp.concatenate([um_num, um_cat], axis=0)


def kernel(x_numerical, x_categorical_idx, x_categorical_meta, W_num, b_num,
           table, W_meta, b_meta, feature_identity, gamma, beta):
    f32 = jnp.float32
    idx = x_categorical_idx.astype(jnp.int32)

    xT = jnp.transpose(x_numerical, (2, 1, 0)).reshape(3 * N_NUM, B)
    idxT = jnp.transpose(idx, (1, 0))
    metaT = jnp.transpose(x_categorical_meta, (1, 2, 0)).reshape(2 * N_CAT, B)

    idx_pad = jnp.pad(idxT, ((0, 6), (0, 0)))
    idx_lin = (idx_pad.reshape(4, 8, 128, 128)
               .transpose(0, 2, 1, 3).reshape(4 * 128 * 8 * 128))
    emb4 = _sc_gather(table, idx_lin)
    embT = (emb4.transpose(0, 2, 1, 3)
            .reshape(N_CAT * D, B))

    WnT = jnp.transpose(W_num, (1, 0))
    WmT = jnp.transpose(W_meta, (1, 0))
    bn = b_num.reshape(D, 1)
    bm = b_meta.reshape(D, 1)
    fid = feature_identity.reshape(NF, D, 1)
    gam = gamma.reshape(D, 1)
    bet = beta.reshape(D, 1)

    grid = (B // BL,)
    lane = lambda rows: pl.BlockSpec((rows, BL), lambda i: (0, i))
    full = lambda shp: pl.BlockSpec(shp, lambda i: tuple(0 for _ in shp))
    h0T, rawT, maskT, umT = pl.pallas_call(
        _tc_body,
        grid=grid,
        in_specs=[
            lane(3 * N_NUM), lane(N_CAT), lane(2 * N_CAT), lane(N_CAT * D),
            full(WnT.shape), full(WmT.shape), full(bn.shape), full(bm.shape),
            full(fid.shape), full(gam.shape), full(bet.shape),
        ],
        out_specs=[lane(NF * D), lane(NF), lane(NF), lane(2 * NF)],
        out_shape=[
            jax.ShapeDtypeStruct((NF * D, B), f32),
            jax.ShapeDtypeStruct((NF, B), f32),
            jax.ShapeDtypeStruct((NF, B), f32),
            jax.ShapeDtypeStruct((2 * NF, B), f32),
        ],
        compiler_params=pltpu.CompilerParams(
            dimension_semantics=("parallel",)),
    )(xT, idxT, metaT, embT, WnT, WmT, bn, bm, fid, gam, bet)

    h_0 = jnp.transpose(h0T.reshape(NF, D, B), (2, 0, 1))
    raw = jnp.transpose(rawT, (1, 0)).reshape(B, NF, 1)
    mask = jnp.transpose(maskT, (1, 0))
    um = jnp.transpose(umT.reshape(NF, 2, B), (2, 0, 1))
    return (h_0, raw, mask, um)


# speedup vs baseline: 1.1013x; 1.1013x over previous
import functools

import jax
import jax.numpy as jnp
import numpy as np
from jax import lax
from jax.experimental import pallas as pl
from jax.experimental.pallas import tpu as pltpu
from jax.experimental.pallas import tpu_sc as plsc

B, N_NUM, N_CAT, D, V = 16384, 13, 26, 32, 1000000
NF = N_NUM + N_CAT
TOT = B * N_CAT
NW = 32
PER_W = TOT // NW
CH = 128
K = 8
GROUPS = PER_W // (CH * K)
BL = 1024


def _sc_gather(table, idx_lin):
    mesh = plsc.VectorSubcoreMesh(core_axis_name="c", subcore_axis_name="s")

    @functools.partial(
        pl.kernel,
        mesh=mesh,
        out_type=jax.ShapeDtypeStruct((TOT, D), jnp.float32),
        scratch_types=[
            pltpu.VMEM((4, 4, 1024), jnp.int32),
            pltpu.VMEM((1024, D), jnp.float32),
            pltpu.SemaphoreType.DMA,
        ],
        compiler_params=pltpu.CompilerParams(use_tc_tiling_on_sc=False),
    )
    def gather_kernel(table_hbm, idx_hbm, out_hbm, idx_v, rows_v, sem):
        wid = lax.axis_index("s") * 2 + lax.axis_index("c")
        for kb in range(4):
            for tc in range(4):
                pltpu.sync_copy(
                    idx_hbm.at[pl.ds((tc * 128 + wid * 4 + kb) * 8 * 128, 1024)],
                    idx_v.at[kb, tc],
                )

        def group(g, carry):
            handles = []
            for half in range(2):
                c = g * 2 + half
                tc = c // 8
                i = c % 8
                for kb in range(4):
                    handles.append(
                        pltpu.async_copy(
                            table_hbm.at[idx_v.at[kb, tc, pl.ds(i * CH, CH)]],
                            rows_v.at[pl.ds((half * 4 + kb) * CH, CH)],
                            sem,
                        )
                    )
            for h in handles:
                h.wait()
            for half in range(2):
                c = g * 2 + half
                pltpu.sync_copy(
                    rows_v.at[pl.ds(half * 512, 512)],
                    out_hbm.at[pl.ds(c * B + wid * 512, 512)],
                )
            return carry

        lax.fori_loop(0, N_CAT // 2, group, 0)

    return gather_kernel(table, idx_lin)


def _tc_body(xT_r, idxT_r, metaT_r, embT_r, WnT_r, WmT_r, bn_r, bm_r,
             fid_r, g_r, bt_r, h0_o, raw_o, mask_o, um_o):
    f32 = jnp.float32
    gelu = lambda t: 0.5 * t * (1.0 + lax.erf(t * np.float32(0.7071067811865476)))

    x3 = xT_r[...].reshape(3, N_NUM, BL)
    m3 = metaT_r[...].reshape(N_CAT, 2, BL)
    e3 = embT_r[...].reshape(N_CAT, D, BL)
    idxv = idxT_r[...]
    fid = fid_r[...]
    gam = g_r[...][None]
    bet = bt_r[...][None]

    wcol = lambda ref, j: ref[:, j:j + 1][None]

    zn = (x3[0][:, None, :] * wcol(WnT_r, 0)
          + x3[1][:, None, :] * wcol(WnT_r, 1)
          + x3[2][:, None, :] * wcol(WnT_r, 2)
          + bn_r[...][None])
    zn = gelu(zn) + fid[:N_NUM]

    zc = (e3
          + m3[:, 0:1, :] * wcol(WmT_r, 0)
          + m3[:, 1:2, :] * wcol(WmT_r, 1)
          + bm_r[...][None])
    zc = gelu(zc) + fid[N_NUM:]

    def ln(z):
        mean = jnp.mean(z, axis=1, keepdims=True)
        var = jnp.mean(z * z, axis=1, keepdims=True) - mean * mean
        return (z - mean) * lax.rsqrt(var + np.float32(1e-5)) * gam + bet

    h0_o[:N_NUM * D, :] = ln(zn).reshape(N_NUM * D, BL)
    h0_o[N_NUM * D:, :] = ln(zc).reshape(N_CAT * D, BL)

    idx_f = idxv.astype(f32)
    raw_o[...] = jnp.concatenate([x3[0], idx_f], axis=0)

    mask_o[...] = jnp.where(
        lax.broadcasted_iota(jnp.int32, (NF, BL), 0) < N_NUM,
        np.float32(1.0), np.float32(0.0))

    um_num = jnp.concatenate(
        [x3[1][:, None, :], x3[2][:, None, :]], axis=1).reshape(2 * N_NUM, BL)
    sign = (idxv % 2 * 2 - 1).astype(f32)
    tf = 0.5 + sign * (0.5 - 0.5 * m3[:, 0, :])
    um_cat = jnp.concatenate(
        [tf[:, None, :], m3[:, 1, :][:, None, :]], axis=1).reshape(2 * N_CAT, BL)
    um_o[...] = jnp.concatenate([um_num, um_cat], axis=0)


def kernel(x_numerical, x_categorical_idx, x_categorical_meta, W_num, b_num,
           table, W_meta, b_meta, feature_identity, gamma, beta):
    f32 = jnp.float32
    idx = x_categorical_idx.astype(jnp.int32)

    xT = jnp.transpose(x_numerical, (2, 1, 0)).reshape(3 * N_NUM, B)
    idxT = jnp.transpose(idx, (1, 0))
    metaT = jnp.transpose(x_categorical_meta, (1, 2, 0)).reshape(2 * N_CAT, B)

    idx_pad = jnp.pad(idxT, ((0, 6), (0, 0)))
    idx_lin = (idx_pad.reshape(4, 8, 128, 128)
               .transpose(0, 2, 1, 3).reshape(4 * 128 * 8 * 128))
    emb = _sc_gather(table, idx_lin)
    embT = jnp.transpose(emb.reshape(N_CAT, B, D), (0, 2, 1)).reshape(N_CAT * D, B)

    WnT = jnp.transpose(W_num, (1, 0))
    WmT = jnp.transpose(W_meta, (1, 0))
    bn = b_num.reshape(D, 1)
    bm = b_meta.reshape(D, 1)
    fid = feature_identity.reshape(NF, D, 1)
    gam = gamma.reshape(D, 1)
    bet = beta.reshape(D, 1)

    grid = (B // BL,)
    lane = lambda rows: pl.BlockSpec((rows, BL), lambda i: (0, i))
    full = lambda shp: pl.BlockSpec(shp, lambda i: tuple(0 for _ in shp))
    h0T, rawT, maskT, umT = pl.pallas_call(
        _tc_body,
        grid=grid,
        in_specs=[
            lane(3 * N_NUM), lane(N_CAT), lane(2 * N_CAT), lane(N_CAT * D),
            full(WnT.shape), full(WmT.shape), full(bn.shape), full(bm.shape),
            full(fid.shape), full(gam.shape), full(bet.shape),
        ],
        out_specs=[lane(NF * D), lane(NF), lane(NF), lane(2 * NF)],
        out_shape=[
            jax.ShapeDtypeStruct((NF * D, B), f32),
            jax.ShapeDtypeStruct((NF, B), f32),
            jax.ShapeDtypeStruct((NF, B), f32),
            jax.ShapeDtypeStruct((2 * NF, B), f32),
        ],
        compiler_params=pltpu.CompilerParams(
            dimension_semantics=("parallel",)),
    )(xT, idxT, metaT, embT, WnT, WmT, bn, bm, fid, gam, bet)

    h_0 = jnp.transpose(h0T.reshape(NF, D, B), (2, 0, 1))
    raw = jnp.transpose(rawT, (1, 0)).reshape(B, NF, 1)
    mask = jnp.transpose(maskT, (1, 0))
    um = jnp.transpose(umT.reshape(NF, 2, B), (2, 0, 1))
    return (h_0, raw, mask, um)


# speedup vs baseline: 1.1963x; 1.0863x over previous
import functools

import jax
import jax.numpy as jnp
import numpy as np
from jax import lax
from jax.experimental import pallas as pl
from jax.experimental.pallas import tpu as pltpu
from jax.experimental.pallas import tpu_sc as plsc

B, N_NUM, N_CAT, D, V = 16384, 13, 26, 32, 1000000
NF = N_NUM + N_CAT
TOT = B * N_CAT
NW = 32
PER_W = TOT // NW
CH = 128
K = 8
GROUPS = PER_W // (CH * K)
BL = 512


def _sc_gather(table, idx_flat):
    idx3 = idx_flat.reshape(NW, PER_W // CH, CH)
    mesh = plsc.VectorSubcoreMesh(core_axis_name="c", subcore_axis_name="s")

    @functools.partial(
        pl.kernel,
        mesh=mesh,
        out_type=jax.ShapeDtypeStruct((TOT, D), jnp.float32),
        scratch_types=[
            pltpu.VMEM((PER_W // CH, CH), jnp.int32),
            pltpu.VMEM((K * CH, D), jnp.float32),
            pltpu.SemaphoreType.DMA,
        ],
        compiler_params=pltpu.CompilerParams(use_tc_tiling_on_sc=False),
    )
    def gather_kernel(table_hbm, idx_hbm, out_hbm, idx_v, rows_v, sem):
        wid = lax.axis_index("s") * 2 + lax.axis_index("c")
        pltpu.sync_copy(idx_hbm.at[wid], idx_v)
        base = wid * PER_W

        def group(g, carry):
            handles = []
            for b in range(K):
                handles.append(
                    pltpu.async_copy(
                        table_hbm.at[idx_v.at[g * K + b]],
                        rows_v.at[pl.ds(b * CH, CH)],
                        sem,
                    )
                )
            for h in handles:
                h.wait()
            pltpu.sync_copy(rows_v, out_hbm.at[pl.ds(base + g * (K * CH), K * CH)])
            return carry

        lax.fori_loop(0, GROUPS, group, 0)

    return gather_kernel(table, idx3)


def _tc_body(xf_r, idx_r, meta_r, emb_r, A_r, Wm_r, Mn_r, En_r, Mc_r, Ec_r,
             R_r, S0_r, S12_r, bnum_r, bcat_r, fidn_r, fidc_r, gn_r, btn_r,
             gc_r, btc_r, h0_o, raw_o, mask_o, um_o):
    f32 = jnp.float32
    dot = lambda a, b: lax.dot_general(
        a, b, (((1,), (0,)), ((), ())), preferred_element_type=f32)
    gelu = lambda t: 0.5 * t * (1.0 + lax.erf(t * np.float32(0.7071067811865476)))

    xf = xf_r[...]
    idxv = idx_r[...]
    m = meta_r[...]

    zn = gelu(dot(xf, A_r[...]) + bnum_r[...]) + fidn_r[...]
    zc = gelu(emb_r[...] + dot(m, Wm_r[...]) + bcat_r[...]) + fidc_r[...]

    def ln(z, M, E, g, bta):
        mean = dot(dot(z, M), E)
        msq = dot(dot(z * z, M), E)
        var = msq - mean * mean
        return (z - mean) * lax.rsqrt(var + np.float32(1e-5)) * g + bta

    h0_o[:, :N_NUM * D] = ln(zn, Mn_r[...], En_r[...], gn_r[...], btn_r[...])
    h0_o[:, N_NUM * D:] = ln(zc, Mc_r[...], Ec_r[...], gc_r[...], btc_r[...])

    idx_f = idxv.astype(f32)
    raw_o[:, :N_NUM] = dot(xf, S0_r[...])
    raw_o[:, N_NUM:] = idx_f

    mask_o[...] = jnp.where(
        lax.broadcasted_iota(jnp.int32, (BL, NF), 1) < N_NUM,
        np.float32(1.0), np.float32(0.0))

    um_o[:, :2 * N_NUM] = dot(xf, S12_r[...])
    sign = (idxv % 2 * 2 - 1).astype(f32)
    s_il = dot(sign, R_r[...])
    even = lax.broadcasted_iota(jnp.int32, (BL, 2 * N_CAT), 1) % 2 == 0
    um_o[:, 2 * N_NUM:] = jnp.where(even, 0.5 + s_il * (0.5 - 0.5 * m), m)


def kernel(x_numerical, x_categorical_idx, x_categorical_meta, W_num, b_num,
           table, W_meta, b_meta, feature_identity, gamma, beta):
    f32 = jnp.float32
    idx = x_categorical_idx.astype(jnp.int32)

    emb = _sc_gather(table, idx.reshape(TOT))

    rn = np.arange(3 * N_NUM)
    cn = np.arange(N_NUM * D)
    seln = jnp.asarray((rn[:, None] // 3 == cn[None, :] // D), f32)
    A = W_num[rn % 3][:, cn % D] * seln
    rc = np.arange(2 * N_CAT)
    cc = np.arange(N_CAT * D)
    selc = jnp.asarray((rc[:, None] // 2 == cc[None, :] // D), f32)
    Wm = W_meta[rc % 2][:, cc % D] * selc

    Mn = jnp.asarray((cn[:, None] // D == np.arange(N_NUM)[None, :]) / D, f32)
    En = jnp.asarray((np.arange(N_NUM)[:, None] == cn[None, :] // D), f32)
    Mc = jnp.asarray((cc[:, None] // D == np.arange(N_CAT)[None, :]) / D, f32)
    Ec = jnp.asarray((np.arange(N_CAT)[:, None] == cc[None, :] // D), f32)

    R = np.zeros((N_CAT, 2 * N_CAT), np.float32)
    R[np.arange(N_CAT), 2 * np.arange(N_CAT)] = 1.0
    R = jnp.asarray(R)
    S0 = np.zeros((3 * N_NUM, N_NUM), np.float32)
    S0[3 * np.arange(N_NUM), np.arange(N_NUM)] = 1.0
    S0 = jnp.asarray(S0)
    S12 = np.zeros((3 * N_NUM, 2 * N_NUM), np.float32)
    S12[3 * np.arange(N_NUM) + 1, 2 * np.arange(N_NUM)] = 1.0
    S12[3 * np.arange(N_NUM) + 2, 2 * np.arange(N_NUM) + 1] = 1.0
    S12 = jnp.asarray(S12)

    row = lambda v: v.reshape(1, -1)
    bnum = row(jnp.tile(b_num, N_NUM))
    bcat = row(jnp.tile(b_meta, N_CAT))
    fid = feature_identity.reshape(NF * D)
    fidn = row(fid[:N_NUM * D])
    fidc = row(fid[N_NUM * D:])
    gn = row(jnp.tile(gamma, N_NUM))
    btn = row(jnp.tile(beta, N_NUM))
    gc = row(jnp.tile(gamma, N_CAT))
    btc = row(jnp.tile(beta, N_CAT))

    xf = x_numerical.reshape(B, 3 * N_NUM)
    meta = x_categorical_meta.reshape(B, 2 * N_CAT)
    emb_f = emb.reshape(B, N_CAT * D)

    grid = (B // BL,)
    bspec = lambda shp, blocked: pl.BlockSpec(
        shp, (lambda i: (i, 0)) if blocked else (lambda i: (0, 0)))
    in_specs = [
        bspec((BL, 3 * N_NUM), True),
        bspec((BL, N_CAT), True),
        bspec((BL, 2 * N_CAT), True),
        bspec((BL, N_CAT * D), True),
        bspec(A.shape, False), bspec(Wm.shape, False),
        bspec(Mn.shape, False), bspec(En.shape, False),
        bspec(Mc.shape, False), bspec(Ec.shape, False),
        bspec(R.shape, False), bspec(S0.shape, False), bspec(S12.shape, False),
        bspec(bnum.shape, False), bspec(bcat.shape, False),
        bspec(fidn.shape, False), bspec(fidc.shape, False),
        bspec(gn.shape, False), bspec(btn.shape, False),
        bspec(gc.shape, False), bspec(btc.shape, False),
    ]
    out_specs = [
        bspec((BL, NF * D), True),
        bspec((BL, NF), True),
        bspec((BL, NF), True),
        bspec((BL, 2 * NF), True),
    ]
    out_shapes = [
        jax.ShapeDtypeStruct((B, NF * D), f32),
        jax.ShapeDtypeStruct((B, NF), f32),
        jax.ShapeDtypeStruct((B, NF), f32),
        jax.ShapeDtypeStruct((B, 2 * NF), f32),
    ]
    h0, raw, mask, um = pl.pallas_call(
        _tc_body,
        grid=grid,
        in_specs=in_specs,
        out_specs=out_specs,
        out_shape=out_shapes,
        compiler_params=pltpu.CompilerParams(
            dimension_semantics=("parallel",)),
    )(xf, idx, meta, emb_f, A, Wm, Mn, En, Mc, Ec, R, S0, S12,
      bnum, bcat, fidn, fidc, gn, btn, gc, btc)

    return (h0.reshape(B, NF, D), raw.reshape(B, NF, 1), mask,
            um.reshape(B, NF, 2))
